# Initial kernel scaffold; baseline (speedup 1.0000x reference)
import functools

import jax
import jax.numpy as jnp
from jax import lax
from jax.experimental import pallas as pl
from jax.experimental.pallas import tpu as pltpu
from jax.experimental.pallas import tpu_sc as plsc

_NKS = 256
_K = 16
_NEG = -1e30

_TOKENS = 4096
_DV = 1024
_R = 256
_NW = 32
_TPW = _TOKENS // _NW


def _topk16(s, iota):
    vals = []
    idxs = []
    for _ in range(_K):
        m = jnp.max(s, axis=1, keepdims=True)
        is_m = s >= m
        p = jnp.sum(jnp.where(is_m, iota, 0.0), axis=1, keepdims=True)
        vals.append(m)
        idxs.append(p)
        s = jnp.where(is_m, _NEG, s)
    return jnp.concatenate(vals, axis=1), jnp.concatenate(idxs, axis=1)


def _tc_body(x_ref, k1_ref, k2_ref, w_ref, idx_ref):
    d2 = x_ref.shape[1] // 2
    x1 = x_ref[:, :d2]
    x2 = x_ref[:, d2:]
    dn = (((1,), (1,)), ((), ()))
    s1 = lax.dot_general(x1, k1_ref[...], dn, preferred_element_type=jnp.float32)
    s2 = lax.dot_general(x2, k2_ref[...], dn, preferred_element_type=jnp.float32)
    iota = lax.broadcasted_iota(jnp.float32, s1.shape, 1)
    v1, i1 = _topk16(s1, iota)
    v2, i2 = _topk16(s2, iota)
    prod = jnp.concatenate([v1[:, i:i + 1] * v2 for i in range(_K)], axis=1)
    vp, ip = _topk16(prod, iota)
    sel_i = jnp.floor(ip * (1.0 / _K))
    sel_j = ip - sel_i * _K
    g1 = jnp.zeros_like(ip)
    g2 = jnp.zeros_like(ip)
    for k in range(_K):
        g1 = g1 + jnp.where(sel_i == k, i1[:, k:k + 1], 0.0)
        g2 = g2 + jnp.where(sel_j == k, i2[:, k:k + 1], 0.0)
    vidx = jnp.clip(g1 * _NKS + g2, 0.0, float(_NKS * _NKS - 1))
    e = jnp.exp(vp - vp[:, 0:1])
    w = e / jnp.sum(e, axis=1, keepdims=True)
    w_ref[...] = w
    idx_ref[...] = vidx.astype(jnp.int32)


def _tc_topk(xf, keys1, keys2, interpret=False):
    return pl.pallas_call(
        _tc_body,
        grid=(_TOKENS // _R,),
        in_specs=[
            pl.BlockSpec((_R, 2 * keys1.shape[1]), lambda i: (i, 0)),
            pl.BlockSpec(keys1.shape, lambda i: (0, 0)),
            pl.BlockSpec(keys2.shape, lambda i: (0, 0)),
        ],
        out_specs=[
            pl.BlockSpec((_R, _K), lambda i: (i, 0)),
            pl.BlockSpec((_R, _K), lambda i: (i, 0)),
        ],
        out_shape=[
            jax.ShapeDtypeStruct((_TOKENS, _K), jnp.float32),
            jax.ShapeDtypeStruct((_TOKENS, _K), jnp.int32),
        ],
        interpret=interpret,
    )(xf, keys1, keys2)


def _sc_body(w_hbm, idx_hbm, values_hbm, y_hbm,
             idx_v, w_v, rows_a, rows_b, out_v, sem_a, sem_b):
    cid = lax.axis_index("c")
    sid = lax.axis_index("s")
    wid = sid * 2 + cid
    base = wid * _TPW
    pltpu.sync_copy(idx_hbm.at[pl.ds(base * _K, _TPW * _K)], idx_v)
    pltpu.sync_copy(w_hbm.at[pl.ds(base, _TPW)], w_v)

    def issue(t, buf, sem):
        ivec = idx_v[pl.ds(t * _K, _K)]
        pltpu.async_copy(values_hbm.at[ivec], buf, sem)

    def drain(buf, sem):
        pltpu.make_async_copy(values_hbm.at[idx_v[pl.ds(0, _K)]], buf, sem).wait()

    def compute(t, buf):
        ws = [w_v[t, k] for k in range(_K)]
        tm8 = lax.rem(t, 8)

        def dbody(d, _):
            s = pl.ds(d * _K, _K)
            terms = [ws[k] * buf[k, s] for k in range(_K)]
            while len(terms) > 1:
                terms = [terms[i] + terms[i + 1] for i in range(0, len(terms), 2)]
            out_v[tm8, s] = terms[0]
            return 0

        lax.fori_loop(0, _DV // _K, dbody, 0)

    n_pairs = _TPW // 2

    def pair(p, _):
        t0 = 2 * p
        t1 = t0 + 1
        issue(t1, rows_b, sem_b)
        drain(rows_a, sem_a)
        compute(t0, rows_a)

        @pl.when(p < n_pairs - 1)
        def _():
            issue(t0 + 2, rows_a, sem_a)

        drain(rows_b, sem_b)
        compute(t1, rows_b)

        @pl.when(lax.rem(p, 4) == 3)
        def _():
            pltpu.sync_copy(out_v, y_hbm.at[pl.ds(base + t1 - 7, 8)])

        return 0

    issue(0, rows_a, sem_a)
    lax.fori_loop(0, n_pairs, pair, 0)


def _sc_gather(w, idx_flat, values):
    mesh = plsc.VectorSubcoreMesh(core_axis_name="c", subcore_axis_name="s")
    fn = pl.kernel(
        _sc_body,
        out_type=jax.ShapeDtypeStruct((_TOKENS, _DV), jnp.float32),
        mesh=mesh,
        scratch_types=[
            pltpu.VMEM((_TPW * _K,), jnp.int32),
            pltpu.VMEM((_TPW, _K), jnp.float32),
            pltpu.VMEM((_K, _DV), jnp.float32),
            pltpu.VMEM((_K, _DV), jnp.float32),
            pltpu.VMEM((8, _DV), jnp.float32),
            pltpu.SemaphoreType.DMA,
            pltpu.SemaphoreType.DMA,
        ],
    )
    return fn(w, idx_flat, values)


def kernel(x, keys1, keys2, values):
    b, s, d = x.shape
    xf = x.reshape(b * s, d)
    w, vidx = _tc_topk(xf, keys1, keys2)
    y = _sc_gather(w, vidx.reshape(-1), values)
    return y.reshape(b, s, values.shape[1])


# speedup vs baseline: 6.2780x; 6.2780x over previous
import functools

import jax
import jax.numpy as jnp
from jax import lax
from jax.experimental import pallas as pl
from jax.experimental.pallas import tpu as pltpu
from jax.experimental.pallas import tpu_sc as plsc

_NKS = 256
_K = 16
_NEG = -1e30

_TOKENS = 4096
_DV = 1024
_R = 1024
_NW = 32
_TPW = _TOKENS // _NW


def _topk16(s, rev_iota):
    vals = []
    idxs = []
    for _ in range(_K):
        m = jnp.max(s, axis=1, keepdims=True)
        masked_rev = jnp.where(s >= m, rev_iota, -1.0)
        p = jnp.max(masked_rev, axis=1, keepdims=True)
        vals.append(m)
        idxs.append(float(_NKS - 1) - p)
        s = jnp.where(masked_rev == p, _NEG, s)
    return jnp.concatenate(vals, axis=1), jnp.concatenate(idxs, axis=1)


def _mm_bf16(x, k):
    dn = (((1,), (1,)), ((), ()))
    return lax.dot_general(x.astype(jnp.bfloat16), k.astype(jnp.bfloat16),
                           dn, preferred_element_type=jnp.float32)


def _tc_body(x_ref, k1_ref, k2_ref, w_ref, idx_ref):
    d2 = x_ref.shape[1] // 2
    x1 = x_ref[:, :d2]
    x2 = x_ref[:, d2:]
    s1 = _mm_bf16(x1, k1_ref[...])
    s2 = _mm_bf16(x2, k2_ref[...])
    rev_iota = float(_NKS - 1) - lax.broadcasted_iota(
        jnp.int32, s1.shape, 1).astype(jnp.float32)
    v1, i1 = _topk16(s1, rev_iota)
    v2, i2 = _topk16(s2, rev_iota)
    prod = jnp.concatenate([v1[:, i:i + 1] * v2 for i in range(_K)], axis=1)
    vp, ip = _topk16(prod, rev_iota)
    sel_i = jnp.floor(ip * (1.0 / _K))
    sel_j = ip - sel_i * _K
    g1 = jnp.zeros_like(ip)
    g2 = jnp.zeros_like(ip)
    for k in range(_K):
        g1 = g1 + jnp.where(sel_i == k, i1[:, k:k + 1], 0.0)
        g2 = g2 + jnp.where(sel_j == k, i2[:, k:k + 1], 0.0)
    vidx = jnp.clip(g1 * _NKS + g2, 0.0, float(_NKS * _NKS - 1))
    e = jnp.exp(vp - vp[:, 0:1])
    w = e / jnp.sum(e, axis=1, keepdims=True)
    w_ref[...] = w
    idx_ref[...] = vidx.astype(jnp.int32)


def _tc_topk(xf, keys1, keys2, interpret=False):
    return pl.pallas_call(
        _tc_body,
        grid=(_TOKENS // _R,),
        in_specs=[
            pl.BlockSpec((_R, 2 * keys1.shape[1]), lambda i: (i, 0)),
            pl.BlockSpec(keys1.shape, lambda i: (0, 0)),
            pl.BlockSpec(keys2.shape, lambda i: (0, 0)),
        ],
        out_specs=[
            pl.BlockSpec((_R, _K), lambda i: (i, 0)),
            pl.BlockSpec((_R, _K), lambda i: (i, 0)),
        ],
        out_shape=[
            jax.ShapeDtypeStruct((_TOKENS, _K), jnp.float32),
            jax.ShapeDtypeStruct((_TOKENS, _K), jnp.int32),
        ],
        interpret=interpret,
    )(xf, keys1, keys2)


def _sc_body(w_hbm, idx_hbm, values_hbm, y_hbm,
             idx_v, w_v, rows_a, rows_b, out_v, sem_a, sem_b):
    cid = lax.axis_index("c")
    sid = lax.axis_index("s")
    wid = sid * 2 + cid
    base = wid * _TPW
    pltpu.sync_copy(idx_hbm.at[pl.ds(base * _K, _TPW * _K)], idx_v)
    pltpu.sync_copy(w_hbm.at[pl.ds(base, _TPW)], w_v)

    def issue(t, buf, sem):
        ivec = idx_v[pl.ds(t * _K, _K)]
        pltpu.async_copy(values_hbm.at[ivec], buf, sem)

    def drain(buf, sem):
        pltpu.make_async_copy(values_hbm.at[idx_v[pl.ds(0, _K)]], buf, sem).wait()

    def compute(t, buf):
        wrow = w_v[t, :]
        ws = [wrow[k] for k in range(_K)]
        tm8 = lax.rem(t, 8)

        def dbody(d, _):
            s = pl.ds(d * _K, _K)
            terms = [ws[k] * buf[k, s] for k in range(_K)]
            while len(terms) > 1:
                terms = [terms[i] + terms[i + 1] for i in range(0, len(terms), 2)]
            out_v[tm8, s] = terms[0]
            return 0

        lax.fori_loop(0, _DV // _K, dbody, 0)

    n_pairs = _TPW // 2

    def pair(p, _):
        t0 = 2 * p
        t1 = t0 + 1
        issue(t1, rows_b, sem_b)
        drain(rows_a, sem_a)
        compute(t0, rows_a)

        @pl.when(p < n_pairs - 1)
        def _():
            issue(t0 + 2, rows_a, sem_a)

        drain(rows_b, sem_b)
        compute(t1, rows_b)

        @pl.when(lax.rem(p, 4) == 3)
        def _():
            off = pl.multiple_of(base + t1 - 7, 8)
            pltpu.sync_copy(out_v, y_hbm.at[pl.ds(off, 8)])

        return 0

    issue(0, rows_a, sem_a)
    lax.fori_loop(0, n_pairs, pair, 0)


def _sc_gather(w, idx_flat, values):
    mesh = plsc.VectorSubcoreMesh(core_axis_name="c", subcore_axis_name="s")
    fn = pl.kernel(
        _sc_body,
        out_type=jax.ShapeDtypeStruct((_TOKENS, _DV), jnp.float32),
        mesh=mesh,
        scratch_types=[
            pltpu.VMEM((_TPW * _K,), jnp.int32),
            pltpu.VMEM((_TPW, _K), jnp.float32),
            pltpu.VMEM((_K, _DV), jnp.float32),
            pltpu.VMEM((_K, _DV), jnp.float32),
            pltpu.VMEM((8, _DV), jnp.float32),
            pltpu.SemaphoreType.DMA,
            pltpu.SemaphoreType.DMA,
        ],
    )
    return fn(w, idx_flat, values)


def kernel(x, keys1, keys2, values):
    b, s, d = x.shape
    xf = x.reshape(b * s, d)
    w, vidx = _tc_topk(xf, keys1, keys2)
    y = _sc_gather(w, vidx.reshape(-1), values)
    return y.reshape(b, s, values.shape[1])
